# baseline (speedup 1.0000x reference)
import functools
import math

import jax
import jax.numpy as jnp
from jax import lax
from jax.experimental import pallas as pl
from jax.experimental.pallas import tpu as pltpu
from jax.experimental.pallas import tpu_sc as plsc

D_MODEL = 64
SCALE = math.sqrt(D_MODEL)
G = 128
C = 1024
LANES = 16


def _build(N):
    NW = 32
    n_w = N // NW
    n_chunks = n_w // C
    k_sub = C // G
    mesh = plsc.VectorSubcoreMesh(core_axis_name="c", subcore_axis_name="s")

    @functools.partial(
        pl.kernel,
        mesh=mesh,
        out_type=jax.ShapeDtypeStruct((N, D_MODEL), jnp.float32),
        compiler_params=pltpu.CompilerParams(use_tc_tiling_on_sc=False),
        scratch_types=[
            pltpu.VMEM((k_sub, G), jnp.int32),
            pltpu.VMEM((C, D_MODEL), jnp.float32),
            pltpu.SemaphoreType.DMA,
        ],
    )
    def emb(x_hbm, table_hbm, out_hbm, idx_v, rows_v, sem):
        cid = lax.axis_index("c")
        sid = lax.axis_index("s")
        wid = sid * 2 + cid
        base = wid * n_w

        def chunk_body(g, carry):
            row0 = pl.multiple_of(base + g * C, C)
            idx0 = pl.multiple_of(base // G + g * k_sub, k_sub)
            pltpu.sync_copy(x_hbm.at[pl.ds(idx0, k_sub)], idx_v)
            cps = [
                pltpu.async_copy(
                    table_hbm.at[idx_v.at[j]],
                    rows_v.at[pl.ds(j * G, G)],
                    sem,
                )
                for j in range(k_sub)
            ]
            for cp in cps:
                cp.wait()

            def scale_row(r, c2):
                for q in range(D_MODEL // LANES):
                    rows_v[r, pl.ds(q * LANES, LANES)] = (
                        rows_v[r, pl.ds(q * LANES, LANES)] * SCALE
                    )
                return c2

            lax.fori_loop(0, C, scale_row, 0, unroll=2)
            pltpu.sync_copy(rows_v, out_hbm.at[pl.ds(row0, C)])
            return carry

        lax.fori_loop(0, n_chunks, chunk_body, 0)

    return emb


def kernel(x, table):
    B, H = x.shape
    N = B * H
    x2 = x.reshape(N // G, G).astype(jnp.int32)
    out = _build(N)(x2, table)
    return out.reshape(B, H, D_MODEL)
